# baseline (speedup 1.0000x reference)
import functools

import jax
import jax.numpy as jnp
from jax import lax
from jax.experimental import pallas as pl
from jax.experimental.pallas import tpu as pltpu
from jax.experimental.pallas import tpu_sc as plsc

NC = 2
NS = 16
NW = NC * NS
K = 128
DEGW = 16


def _ceil_to(x, m):
    return -(-x // m) * m


@functools.lru_cache(maxsize=None)
def _build(n, e, d):
    npad = _ceil_to(n + 1, 256)
    rpt = npad // NW
    nfull, rem = divmod(rpt, 128)
    RF = 16
    epad = _ceil_to(e, NW * K * RF)
    chunks = epad // (NW * K)
    rb = max(b for b in (1024, 512, 256) if npad % b == 0)
    grid = npad // rb
    dl = d // 16

    mesh = plsc.VectorSubcoreMesh(core_axis_name="c", subcore_axis_name="s")

    @functools.partial(
        pl.kernel,
        out_type=jax.ShapeDtypeStruct((NC, npad, DEGW), jnp.float32),
        mesh=mesh,
        scratch_types=[
            pltpu.VMEM((chunks, 2, K), jnp.int32),
            pltpu.VMEM((K, DEGW), jnp.float32),
            pltpu.VMEM((rpt, DEGW), jnp.float32),
            pltpu.VMEM_SHARED((npad, DEGW), jnp.float32),
            pltpu.SemaphoreType.DMA,
            pltpu.SemaphoreType.DMA,
        ],
    )
    def _degree(epk_hbm, out_hbm, epk_v, ones_v, zero_v, acc, sem0, sem1):
        c = lax.axis_index("c")
        s = lax.axis_index("s")
        wid = c * NS + s
        pltpu.sync_copy(epk_hbm.at[wid], epk_v)

        def fill_ones(i, carry):
            ones_v[i, :] = jnp.ones((DEGW,), jnp.float32)
            return carry

        lax.fori_loop(0, K, fill_ones, 0)

        def fill_zero(i, carry):
            zero_v[i, :] = jnp.zeros((DEGW,), jnp.float32)
            return carry

        lax.fori_loop(0, rpt, fill_zero, 0)
        r0 = s * rpt
        pltpu.sync_copy(zero_v, acc.at[pl.ds(r0, rpt), :])
        plsc.subcore_barrier()

        @pl.loop(0, chunks, step=2)
        def _eloop(j):
            d0 = pltpu.async_copy(ones_v, acc.at[epk_v.at[j, 0]], sem0, add=True)
            d1 = pltpu.async_copy(ones_v, acc.at[epk_v.at[j + 1, 0]], sem1, add=True)
            d0.wait()
            d1.wait()

        plsc.subcore_barrier()
        pltpu.sync_copy(acc.at[pl.ds(r0, rpt), :], out_hbm.at[c, pl.ds(r0, rpt), :])

    zrows = 64
    zfull, zrem = divmod(rpt, zrows)

    @functools.partial(
        pl.kernel,
        out_type=jax.ShapeDtypeStruct((NC, npad, d), jnp.float32),
        mesh=mesh,
        scratch_types=[
            pltpu.VMEM((RF, 2, K), jnp.int32),
            pltpu.VMEM((K, d), jnp.float32),
            pltpu.VMEM((K, d), jnp.float32),
            pltpu.VMEM((zrows, d), jnp.float32),
            pltpu.VMEM_SHARED((npad, d), jnp.float32),
            pltpu.SemaphoreType.DMA,
            pltpu.SemaphoreType.DMA,
            pltpu.SemaphoreType.DMA,
            pltpu.SemaphoreType.DMA,
        ],
    )
    def _propagate(h_hbm, epk_hbm, out_hbm, epk_v, rows0, rows1,
                   zero_v, acc, gsem0, gsem1, ssem0, ssem1):
        c = lax.axis_index("c")
        s = lax.axis_index("s")
        wid = c * NS + s

        def fz(i, carry):
            zero_v[i // dl, pl.ds((i % dl) * 16, 16)] = jnp.zeros((16,), jnp.float32)
            return carry

        lax.fori_loop(0, zrows * dl, fz, 0)
        r0 = s * rpt
        for b in range(zfull):
            pltpu.sync_copy(zero_v, acc.at[pl.ds(r0 + b * zrows, zrows), :])
        if zrem:
            pltpu.sync_copy(zero_v.at[pl.ds(0, zrem), :],
                            acc.at[pl.ds(r0 + zfull * zrows, zrem), :])
        plsc.subcore_barrier()

        rows = (rows0, rows1)
        gsem = (gsem0, gsem1)
        ssem = (ssem0, ssem1)

        @pl.loop(0, chunks, step=RF)
        def _blk(j0):
            pltpu.sync_copy(epk_hbm.at[wid, pl.ds(j0, RF)], epk_v)
            pltpu.async_copy(h_hbm.at[epk_v.at[0, 0]], rows0, gsem0)
            for t in range(RF):
                b = t % 2
                pltpu.make_async_copy(
                    h_hbm.at[epk_v.at[t, 0]], rows[b], gsem[b]).wait()
                sd = pltpu.async_copy(
                    rows[b], acc.at[epk_v.at[t, 1]], ssem[b], add=True)
                if t + 1 < RF:
                    pltpu.async_copy(
                        h_hbm.at[epk_v.at[t + 1, 0]], rows[1 - b], gsem[1 - b])
                sd.wait()

        plsc.subcore_barrier()
        for b in range(nfull):
            pltpu.sync_copy(acc.at[pl.ds(r0 + b * 128, 128), :],
                            out_hbm.at[c, pl.ds(r0 + b * 128, 128), :])
        if rem:
            pltpu.sync_copy(acc.at[pl.ds(r0 + nfull * 128, rem), :],
                            out_hbm.at[c, pl.ds(r0 + nfull * 128, rem), :])

    def _fc0_body(x_ref, degs_ref, w_ref, b_ref, h_ref, dis_ref):
        deg2 = degs_ref[0, :, :1] + degs_ref[1, :, :1]
        dis2 = deg2 ** -0.5
        h = jnp.dot(x_ref[...], w_ref[...], preferred_element_type=jnp.float32)
        h_ref[...] = (h + b_ref[...][None, :]) * dis2
        dis_ref[...] = dis2[:, 0]

    _fc0 = pl.pallas_call(
        _fc0_body,
        grid=(grid,),
        in_specs=[
            pl.BlockSpec((rb, d), lambda i: (i, 0)),
            pl.BlockSpec((NC, rb, DEGW), lambda i: (0, i, 0)),
            pl.BlockSpec((d, d), lambda i: (0, 0)),
            pl.BlockSpec((d,), lambda i: (0,)),
        ],
        out_specs=[
            pl.BlockSpec((rb, d), lambda i: (i, 0)),
            pl.BlockSpec((rb,), lambda i: (i,)),
        ],
        out_shape=[
            jax.ShapeDtypeStruct((npad, d), jnp.float32),
            jax.ShapeDtypeStruct((npad,), jnp.float32),
        ],
    )

    def _fc1_body(p_ref, dis_ref, w_ref, b_ref, h_ref):
        dis2 = dis_ref[...][:, None]
        h1 = jnp.maximum(2.0 * dis2 * (p_ref[0] + p_ref[1]), 0.0)
        h = jnp.dot(h1, w_ref[...], preferred_element_type=jnp.float32)
        h_ref[...] = (h + b_ref[...][None, :]) * dis2

    _fc1 = pl.pallas_call(
        _fc1_body,
        grid=(grid,),
        in_specs=[
            pl.BlockSpec((NC, rb, d), lambda i: (0, i, 0)),
            pl.BlockSpec((rb,), lambda i: (i,)),
            pl.BlockSpec((d, d), lambda i: (0, 0)),
            pl.BlockSpec((d,), lambda i: (0,)),
        ],
        out_specs=pl.BlockSpec((rb, d), lambda i: (i, 0)),
        out_shape=jax.ShapeDtypeStruct((npad, d), jnp.float32),
    )

    def _final_body(p_ref, dis_ref, o_ref):
        o_ref[...] = (p_ref[0] + p_ref[1]) * dis_ref[...][:, None]

    _final = pl.pallas_call(
        _final_body,
        grid=(grid,),
        in_specs=[
            pl.BlockSpec((NC, rb, d), lambda i: (0, i, 0)),
            pl.BlockSpec((rb,), lambda i: (i,)),
        ],
        out_specs=pl.BlockSpec((rb, d), lambda i: (i, 0)),
        out_shape=jax.ShapeDtypeStruct((npad, d), jnp.float32),
    )

    def run(xpad, epk, w0t, b0, w1t, b1):
        degs = _degree(epk)
        h0, dis = _fc0(xpad, degs, w0t, b0)
        p0 = _propagate(h0, epk)
        h1 = _fc1(p0, dis, w1t, b1)
        p1 = _propagate(h1, epk)
        return _final(p1, dis)

    return run, npad, epad, chunks


def kernel(inputs, edge_index, W0, b0, W1, b1):
    n, d = inputs.shape
    e = edge_index.shape[1]
    run, npad, epad, chunks = _build(n, e, d)
    src = jnp.concatenate(
        [edge_index[0].astype(jnp.int32), jnp.full((epad - e,), n, jnp.int32)])
    dst = jnp.concatenate(
        [edge_index[1].astype(jnp.int32), jnp.full((epad - e,), n, jnp.int32)])
    epk = jnp.stack([src.reshape(NW, chunks, K), dst.reshape(NW, chunks, K)],
                    axis=2)
    xpad = jnp.pad(inputs.astype(jnp.float32), ((0, npad - n), (0, 0)))
    out = run(xpad, epk, W0.T, b0, W1.T, b1)
    return out[:n]


# speedup vs baseline: 2.7697x; 2.7697x over previous
import functools

import jax
import jax.numpy as jnp
from jax import lax
from jax.experimental import pallas as pl
from jax.experimental.pallas import tpu as pltpu
from jax.experimental.pallas import tpu_sc as plsc

NC = 2
NS = 16
NW = NC * NS
K = 128
DEGW = 16


def _ceil_to(x, m):
    return -(-x // m) * m


@functools.lru_cache(maxsize=None)
def _build(n, e, d):
    npad = _ceil_to(n + 1, 256)
    rpt = npad // NW
    nfull, rem = divmod(rpt, 128)
    RF = 16
    epad = _ceil_to(e, NW * K * RF)
    chunks = epad // (NW * K)
    rb = max(b for b in (1024, 512, 256) if npad % b == 0)
    grid = npad // rb
    dl = d // 16

    mesh = plsc.VectorSubcoreMesh(core_axis_name="c", subcore_axis_name="s")

    @functools.partial(
        pl.kernel,
        out_type=jax.ShapeDtypeStruct((NC, npad, DEGW), jnp.float32),
        mesh=mesh,
        scratch_types=[
            pltpu.VMEM((chunks, 2, K), jnp.int32),
            pltpu.VMEM((K, DEGW), jnp.float32),
            pltpu.VMEM((rpt, DEGW), jnp.float32),
            pltpu.VMEM_SHARED((npad, DEGW), jnp.float32),
            pltpu.SemaphoreType.DMA,
            pltpu.SemaphoreType.DMA,
        ],
    )
    def _degree(epk_hbm, out_hbm, epk_v, ones_v, zero_v, acc, sem0, sem1):
        c = lax.axis_index("c")
        s = lax.axis_index("s")
        wid = c * NS + s
        pltpu.sync_copy(epk_hbm.at[wid], epk_v)

        def fill_ones(i, carry):
            ones_v[i, :] = jnp.ones((DEGW,), jnp.float32)
            return carry

        lax.fori_loop(0, K, fill_ones, 0)

        def fill_zero(i, carry):
            zero_v[i, :] = jnp.zeros((DEGW,), jnp.float32)
            return carry

        lax.fori_loop(0, rpt, fill_zero, 0)
        r0 = s * rpt
        pltpu.sync_copy(zero_v, acc.at[pl.ds(r0, rpt), :])
        plsc.subcore_barrier()

        @pl.loop(0, chunks, step=2)
        def _eloop(j):
            d0 = pltpu.async_copy(ones_v, acc.at[epk_v.at[j, 0]], sem0, add=True)
            d1 = pltpu.async_copy(ones_v, acc.at[epk_v.at[j + 1, 0]], sem1, add=True)
            d0.wait()
            d1.wait()

        plsc.subcore_barrier()
        pltpu.sync_copy(acc.at[pl.ds(r0, rpt), :], out_hbm.at[c, pl.ds(r0, rpt), :])

    zrows = 64
    zfull, zrem = divmod(rpt, zrows)

    @functools.partial(
        pl.kernel,
        out_type=jax.ShapeDtypeStruct((NC, npad, d), jnp.float32),
        mesh=mesh,
        scratch_types=[
            pltpu.VMEM((RF, 2, K), jnp.int32),
            pltpu.VMEM((K, d), jnp.float32),
            pltpu.VMEM((K, d), jnp.float32),
            pltpu.VMEM((zrows, d), jnp.float32),
            pltpu.VMEM_SHARED((npad, d), jnp.float32),
            pltpu.SemaphoreType.DMA,
            pltpu.SemaphoreType.DMA,
            pltpu.SemaphoreType.DMA,
            pltpu.SemaphoreType.DMA,
        ],
    )
    def _propagate(h_hbm, epk_hbm, out_hbm, epk_v, rows0, rows1,
                   zero_v, acc, gsem0, gsem1, ssem0, ssem1):
        c = lax.axis_index("c")
        s = lax.axis_index("s")
        wid = c * NS + s

        def fz(i, carry):
            zero_v[i // dl, pl.ds((i % dl) * 16, 16)] = jnp.zeros((16,), jnp.float32)
            return carry

        lax.fori_loop(0, zrows * dl, fz, 0)
        r0 = s * rpt
        for b in range(zfull):
            pltpu.sync_copy(zero_v, acc.at[pl.ds(r0 + b * zrows, zrows), :])
        if zrem:
            pltpu.sync_copy(zero_v.at[pl.ds(0, zrem), :],
                            acc.at[pl.ds(r0 + zfull * zrows, zrem), :])
        plsc.subcore_barrier()

        rows = (rows0, rows1)
        gsem = (gsem0, gsem1)
        ssem = (ssem0, ssem1)

        @pl.loop(0, chunks, step=RF)
        def _blk(j0):
            pltpu.sync_copy(epk_hbm.at[wid, pl.ds(j0, RF)], epk_v)
            pltpu.async_copy(h_hbm.at[epk_v.at[0, 0]], rows0, gsem0)
            for t in range(RF):
                b = t % 2
                pltpu.make_async_copy(
                    h_hbm.at[epk_v.at[t, 0]], rows[b], gsem[b]).wait()
                sd = pltpu.async_copy(
                    rows[b], acc.at[epk_v.at[t, 1]], ssem[b], add=True)
                if t + 1 < RF:
                    pltpu.async_copy(
                        h_hbm.at[epk_v.at[t + 1, 0]], rows[1 - b], gsem[1 - b])
                sd.wait()

        plsc.subcore_barrier()
        for b in range(nfull):
            pltpu.sync_copy(acc.at[pl.ds(r0 + b * 128, 128), :],
                            out_hbm.at[c, pl.ds(r0 + b * 128, 128), :])
        if rem:
            pltpu.sync_copy(acc.at[pl.ds(r0 + nfull * 128, rem), :],
                            out_hbm.at[c, pl.ds(r0 + nfull * 128, rem), :])

    def _fc0_body(x_ref, degs_ref, w_ref, b_ref, h_ref, dis_ref):
        deg2 = degs_ref[0, :, :1] + degs_ref[1, :, :1]
        dis2 = deg2 ** -0.5
        h = jnp.dot(x_ref[...], w_ref[...], preferred_element_type=jnp.float32)
        h_ref[...] = (h + b_ref[...][None, :]) * dis2
        dis_ref[...] = dis2[:, 0]

    _fc0 = pl.pallas_call(
        _fc0_body,
        grid=(grid,),
        in_specs=[
            pl.BlockSpec((rb, d), lambda i: (i, 0)),
            pl.BlockSpec((NC, rb, DEGW), lambda i: (0, i, 0)),
            pl.BlockSpec((d, d), lambda i: (0, 0)),
            pl.BlockSpec((d,), lambda i: (0,)),
        ],
        out_specs=[
            pl.BlockSpec((rb, d), lambda i: (i, 0)),
            pl.BlockSpec((rb,), lambda i: (i,)),
        ],
        out_shape=[
            jax.ShapeDtypeStruct((npad, d), jnp.float32),
            jax.ShapeDtypeStruct((npad,), jnp.float32),
        ],
    )

    def _fc1_body(p_ref, dis_ref, w_ref, b_ref, h_ref):
        dis2 = dis_ref[...][:, None]
        h1 = jnp.maximum(2.0 * dis2 * (p_ref[0] + p_ref[1]), 0.0)
        h = jnp.dot(h1, w_ref[...], preferred_element_type=jnp.float32)
        h_ref[...] = (h + b_ref[...][None, :]) * dis2

    _fc1 = pl.pallas_call(
        _fc1_body,
        grid=(grid,),
        in_specs=[
            pl.BlockSpec((NC, rb, d), lambda i: (0, i, 0)),
            pl.BlockSpec((rb,), lambda i: (i,)),
            pl.BlockSpec((d, d), lambda i: (0, 0)),
            pl.BlockSpec((d,), lambda i: (0,)),
        ],
        out_specs=pl.BlockSpec((rb, d), lambda i: (i, 0)),
        out_shape=jax.ShapeDtypeStruct((npad, d), jnp.float32),
    )

    def _final_body(p_ref, dis_ref, o_ref):
        o_ref[...] = (p_ref[0] + p_ref[1]) * dis_ref[...][:, None]

    _final = pl.pallas_call(
        _final_body,
        grid=(grid,),
        in_specs=[
            pl.BlockSpec((NC, rb, d), lambda i: (0, i, 0)),
            pl.BlockSpec((rb,), lambda i: (i,)),
        ],
        out_specs=pl.BlockSpec((rb, d), lambda i: (i, 0)),
        out_shape=jax.ShapeDtypeStruct((npad, d), jnp.float32),
    )

    def run(xpad, epk, w0t, b0, w1t, b1):
        degs = _degree(epk)
        h0, dis = _fc0(xpad, degs, w0t, b0)
        p0 = _propagate(h0, epk)
        h1 = _fc1(p0, dis, w1t, b1)
        p1 = _propagate(h1, epk)
        return _final(p1, dis)

    return run, npad, epad, chunks


def kernel(inputs, edge_index, W0, b0, W1, b1):
    n, d = inputs.shape
    e = edge_index.shape[1]
    run, npad, epad, chunks = _build(n, e, d)
    pad_idx = n + (jnp.arange(epad - e, dtype=jnp.int32) % (npad - n))
    src = jnp.concatenate([edge_index[0].astype(jnp.int32), pad_idx])
    dst = jnp.concatenate([edge_index[1].astype(jnp.int32), pad_idx])
    epk = jnp.stack([src.reshape(NW, chunks, K), dst.reshape(NW, chunks, K)],
                    axis=2)
    xpad = jnp.pad(inputs.astype(jnp.float32), ((0, npad - n), (0, 0)))
    out = run(xpad, epk, W0.T, b0, W1.T, b1)
    return out[:n]


# speedup vs baseline: 2.8754x; 1.0382x over previous
import functools

import jax
import jax.numpy as jnp
from jax import lax
from jax.experimental import pallas as pl
from jax.experimental.pallas import tpu as pltpu
from jax.experimental.pallas import tpu_sc as plsc

NC = 2
NS = 16
NW = NC * NS
K = 128
DEGW = 16


def _ceil_to(x, m):
    return -(-x // m) * m


@functools.lru_cache(maxsize=None)
def _build(n, e, d):
    npad = _ceil_to(n + 1, 256)
    rpt = npad // NW
    nfull, rem = divmod(rpt, 128)
    RF = 20
    epad = _ceil_to(e, NW * K * 2 * RF)
    chunks = epad // (NW * K)
    rb = max(b for b in (1024, 512, 256) if npad % b == 0)
    grid = npad // rb
    dl = d // 16

    mesh = plsc.VectorSubcoreMesh(core_axis_name="c", subcore_axis_name="s")

    @functools.partial(
        pl.kernel,
        out_type=jax.ShapeDtypeStruct((NC, npad, DEGW), jnp.float32),
        mesh=mesh,
        scratch_types=[
            pltpu.VMEM((chunks, 2, K), jnp.int32),
            pltpu.VMEM((K, DEGW), jnp.float32),
            pltpu.VMEM((rpt, DEGW), jnp.float32),
            pltpu.VMEM_SHARED((npad, DEGW), jnp.float32),
            pltpu.SemaphoreType.DMA,
            pltpu.SemaphoreType.DMA,
        ],
    )
    def _degree(epk_hbm, out_hbm, epk_v, ones_v, zero_v, acc, sem0, sem1):
        c = lax.axis_index("c")
        s = lax.axis_index("s")
        wid = c * NS + s
        pltpu.sync_copy(epk_hbm.at[wid], epk_v)

        def fill_ones(i, carry):
            ones_v[i, :] = jnp.ones((DEGW,), jnp.float32)
            return carry

        lax.fori_loop(0, K, fill_ones, 0)

        def fill_zero(i, carry):
            zero_v[i, :] = jnp.zeros((DEGW,), jnp.float32)
            return carry

        lax.fori_loop(0, rpt, fill_zero, 0)
        r0 = s * rpt
        pltpu.sync_copy(zero_v, acc.at[pl.ds(r0, rpt), :])
        plsc.subcore_barrier()

        @pl.loop(0, chunks, step=2)
        def _eloop(j):
            d0 = pltpu.async_copy(ones_v, acc.at[epk_v.at[j, 0]], sem0, add=True)
            d1 = pltpu.async_copy(ones_v, acc.at[epk_v.at[j + 1, 0]], sem1, add=True)
            d0.wait()
            d1.wait()

        plsc.subcore_barrier()
        pltpu.sync_copy(acc.at[pl.ds(r0, rpt), :], out_hbm.at[c, pl.ds(r0, rpt), :])

    zrows = 64
    zfull, zrem = divmod(rpt, zrows)

    @functools.partial(
        pl.kernel,
        out_type=jax.ShapeDtypeStruct((NC, npad, d), jnp.float32),
        mesh=mesh,
        scratch_types=[
            pltpu.VMEM((RF, 2, K), jnp.int32),
            pltpu.VMEM((RF, 2, K), jnp.int32),
            pltpu.VMEM((K, d), jnp.float32),
            pltpu.VMEM((K, d), jnp.float32),
            pltpu.VMEM_SHARED((npad, d), jnp.float32),
            pltpu.SemaphoreType.DMA,
            pltpu.SemaphoreType.DMA,
            pltpu.SemaphoreType.DMA,
            pltpu.SemaphoreType.DMA,
            pltpu.SemaphoreType.DMA,
            pltpu.SemaphoreType.DMA,
        ],
    )
    def _propagate(h_hbm, epk_hbm, out_hbm, epk0, epk1, rows0, rows1,
                   acc, gsem0, gsem1, ssem0, ssem1, rsem0, rsem1):
        c = lax.axis_index("c")
        s = lax.axis_index("s")
        wid = c * NS + s

        def fz(i, carry):
            rows0[i // dl, pl.ds((i % dl) * 16, 16)] = jnp.zeros((16,), jnp.float32)
            return carry

        lax.fori_loop(0, zrows * dl, fz, 0)
        r0 = s * rpt
        for b in range(zfull):
            pltpu.sync_copy(rows0.at[pl.ds(0, zrows), :],
                            acc.at[pl.ds(r0 + b * zrows, zrows), :])
        if zrem:
            pltpu.sync_copy(rows0.at[pl.ds(0, zrem), :],
                            acc.at[pl.ds(r0 + zfull * zrows, zrem), :])
        plsc.subcore_barrier()

        rows = (rows0, rows1)
        gsem = (gsem0, gsem1)
        ssem = (ssem0, ssem1)
        epk = (epk0, epk1)
        rsem = (rsem0, rsem1)

        pltpu.sync_copy(epk_hbm.at[wid, pl.ds(0, RF)], epk0)
        pltpu.async_copy(epk_hbm.at[wid, pl.ds(RF, RF)], epk1, rsem1)
        pltpu.async_copy(h_hbm.at[epk0.at[0, 0]], rows0, gsem0)

        @pl.loop(0, chunks, step=2 * RF)
        def _blk(j0):
            for blk in range(2):
                cur = epk[blk]
                nxt = epk[1 - blk]
                base = j0 + blk * RF
                for t in range(RF):
                    b = t % 2
                    pltpu.make_async_copy(
                        h_hbm.at[cur.at[t, 0]], rows[b], gsem[b]).wait()
                    sd = pltpu.async_copy(
                        rows[b], acc.at[cur.at[t, 1]], ssem[b], add=True)
                    if t + 1 < RF:
                        pltpu.async_copy(
                            h_hbm.at[cur.at[t + 1, 0]], rows[1 - b],
                            gsem[1 - b])
                    else:
                        @pl.when(base + RF < chunks)
                        def _():
                            pltpu.make_async_copy(
                                epk_hbm.at[wid, pl.ds(base + RF, RF)],
                                nxt, rsem[1 - blk]).wait()
                            pltpu.async_copy(
                                h_hbm.at[nxt.at[0, 0]], rows[1 - b],
                                gsem[1 - b])
                    sd.wait()

                @pl.when(base + 2 * RF < chunks)
                def _():
                    pltpu.async_copy(
                        epk_hbm.at[wid, pl.ds(base + 2 * RF, RF)],
                        cur, rsem[blk])

        plsc.subcore_barrier()
        for b in range(nfull):
            pltpu.sync_copy(acc.at[pl.ds(r0 + b * 128, 128), :],
                            out_hbm.at[c, pl.ds(r0 + b * 128, 128), :])
        if rem:
            pltpu.sync_copy(acc.at[pl.ds(r0 + nfull * 128, rem), :],
                            out_hbm.at[c, pl.ds(r0 + nfull * 128, rem), :])

    def _mm0_body(x_ref, w_ref, b_ref, o_ref):
        h = jnp.dot(x_ref[...], w_ref[...], preferred_element_type=jnp.float32)
        o_ref[...] = h + b_ref[...][None, :]

    _mm0 = pl.pallas_call(
        _mm0_body,
        grid=(grid,),
        in_specs=[
            pl.BlockSpec((rb, d), lambda i: (i, 0)),
            pl.BlockSpec((d, d), lambda i: (0, 0)),
            pl.BlockSpec((d,), lambda i: (0,)),
        ],
        out_specs=pl.BlockSpec((rb, d), lambda i: (i, 0)),
        out_shape=jax.ShapeDtypeStruct((npad, d), jnp.float32),
    )

    def _scale0_body(xw_ref, degs_ref, h_ref, dis_ref):
        deg2 = degs_ref[0, :, :1] + degs_ref[1, :, :1]
        dis2 = deg2 ** -0.5
        h_ref[...] = xw_ref[...] * dis2
        dis_ref[...] = dis2[:, 0]

    _scale0 = pl.pallas_call(
        _scale0_body,
        grid=(grid,),
        in_specs=[
            pl.BlockSpec((rb, d), lambda i: (i, 0)),
            pl.BlockSpec((NC, rb, DEGW), lambda i: (0, i, 0)),
        ],
        out_specs=[
            pl.BlockSpec((rb, d), lambda i: (i, 0)),
            pl.BlockSpec((rb,), lambda i: (i,)),
        ],
        out_shape=[
            jax.ShapeDtypeStruct((npad, d), jnp.float32),
            jax.ShapeDtypeStruct((npad,), jnp.float32),
        ],
    )

    def _fc1_body(p_ref, dis_ref, w_ref, b_ref, h_ref):
        dis2 = dis_ref[...][:, None]
        h1 = jnp.maximum(2.0 * dis2 * (p_ref[0] + p_ref[1]), 0.0)
        h = jnp.dot(h1, w_ref[...], preferred_element_type=jnp.float32)
        h_ref[...] = (h + b_ref[...][None, :]) * dis2

    _fc1 = pl.pallas_call(
        _fc1_body,
        grid=(grid,),
        in_specs=[
            pl.BlockSpec((NC, rb, d), lambda i: (0, i, 0)),
            pl.BlockSpec((rb,), lambda i: (i,)),
            pl.BlockSpec((d, d), lambda i: (0, 0)),
            pl.BlockSpec((d,), lambda i: (0,)),
        ],
        out_specs=pl.BlockSpec((rb, d), lambda i: (i, 0)),
        out_shape=jax.ShapeDtypeStruct((npad, d), jnp.float32),
    )

    def _final_body(p_ref, dis_ref, o_ref):
        o_ref[...] = (p_ref[0] + p_ref[1]) * dis_ref[...][:, None]

    _final = pl.pallas_call(
        _final_body,
        grid=(grid,),
        in_specs=[
            pl.BlockSpec((NC, rb, d), lambda i: (0, i, 0)),
            pl.BlockSpec((rb,), lambda i: (i,)),
        ],
        out_specs=pl.BlockSpec((rb, d), lambda i: (i, 0)),
        out_shape=jax.ShapeDtypeStruct((npad, d), jnp.float32),
    )

    def run(xpad, epk, w0t, b0, w1t, b1):
        degs = _degree(epk)
        xw0 = _mm0(xpad, w0t, b0)
        h0, dis = _scale0(xw0, degs)
        p0 = _propagate(h0, epk)
        h1 = _fc1(p0, dis, w1t, b1)
        p1 = _propagate(h1, epk)
        return _final(p1, dis)

    return run, npad, epad, chunks


def kernel(inputs, edge_index, W0, b0, W1, b1):
    n, d = inputs.shape
    e = edge_index.shape[1]
    run, npad, epad, chunks = _build(n, e, d)
    pad_idx = n + (jnp.arange(epad - e, dtype=jnp.int32) % (npad - n))
    src = jnp.concatenate([edge_index[0].astype(jnp.int32), pad_idx])
    dst = jnp.concatenate([edge_index[1].astype(jnp.int32), pad_idx])
    epk = jnp.stack([src.reshape(NW, chunks, K), dst.reshape(NW, chunks, K)],
                    axis=2)
    xpad = jnp.pad(inputs.astype(jnp.float32), ((0, npad - n), (0, 0)))
    out = run(xpad, epk, W0.T, b0, W1.T, b1)
    return out[:n]


# speedup vs baseline: 2.8795x; 1.0014x over previous
import functools

import jax
import jax.numpy as jnp
from jax import lax
from jax.experimental import pallas as pl
from jax.experimental.pallas import tpu as pltpu
from jax.experimental.pallas import tpu_sc as plsc

NC = 2
NS = 16
NW = NC * NS
K = 128
DEGW = 16


def _ceil_to(x, m):
    return -(-x // m) * m


@functools.lru_cache(maxsize=None)
def _build(n, e, d):
    npad = _ceil_to(n + 1, 256)
    rpt = npad // NW
    nfull, rem = divmod(rpt, 128)
    RF = 20
    epad = _ceil_to(e, NW * K * 2 * RF)
    chunks = epad // (NW * K)
    rb = max(b for b in (1024, 512, 256) if npad % b == 0)
    grid = npad // rb
    dl = d // 16

    mesh = plsc.VectorSubcoreMesh(core_axis_name="c", subcore_axis_name="s")

    @functools.partial(
        pl.kernel,
        out_type=jax.ShapeDtypeStruct((NC, npad, DEGW), jnp.float32),
        mesh=mesh,
        scratch_types=[
            pltpu.VMEM((chunks, 2, K), jnp.int32),
            pltpu.VMEM((K, DEGW), jnp.float32),
            pltpu.VMEM((rpt, DEGW), jnp.float32),
            pltpu.VMEM_SHARED((npad, DEGW), jnp.float32),
            pltpu.SemaphoreType.DMA,
            pltpu.SemaphoreType.DMA,
        ],
    )
    def _degree(epk_hbm, out_hbm, epk_v, ones_v, zero_v, acc, sem0, sem1):
        c = lax.axis_index("c")
        s = lax.axis_index("s")
        wid = c * NS + s
        pltpu.sync_copy(epk_hbm.at[wid], epk_v)

        def fill_ones(i, carry):
            ones_v[i, :] = jnp.ones((DEGW,), jnp.float32)
            return carry

        lax.fori_loop(0, K, fill_ones, 0)

        def fill_zero(i, carry):
            zero_v[i, :] = jnp.zeros((DEGW,), jnp.float32)
            return carry

        lax.fori_loop(0, rpt, fill_zero, 0)
        r0 = s * rpt
        pltpu.sync_copy(zero_v, acc.at[pl.ds(r0, rpt), :])
        plsc.subcore_barrier()

        @pl.loop(0, chunks, step=2)
        def _eloop(j):
            d0 = pltpu.async_copy(ones_v, acc.at[epk_v.at[j, 0]], sem0, add=True)
            d1 = pltpu.async_copy(ones_v, acc.at[epk_v.at[j + 1, 0]], sem1, add=True)
            d0.wait()
            d1.wait()

        plsc.subcore_barrier()
        pltpu.sync_copy(acc.at[pl.ds(r0, rpt), :], out_hbm.at[c, pl.ds(r0, rpt), :])

    zrows = 64
    zfull, zrem = divmod(rpt, zrows)

    @functools.partial(
        pl.kernel,
        out_type=jax.ShapeDtypeStruct((NC, npad, d), jnp.float32),
        mesh=mesh,
        scratch_types=[
            pltpu.VMEM((RF, 2, K), jnp.int32),
            pltpu.VMEM((RF, 2, K), jnp.int32),
            pltpu.VMEM((K, d), jnp.float32),
            pltpu.VMEM((K, d), jnp.float32),
            pltpu.VMEM_SHARED((npad, d), jnp.float32),
            pltpu.SemaphoreType.DMA,
            pltpu.SemaphoreType.DMA,
            pltpu.SemaphoreType.DMA,
            pltpu.SemaphoreType.DMA,
            pltpu.SemaphoreType.DMA,
            pltpu.SemaphoreType.DMA,
        ],
    )
    def _propagate(h_hbm, epk_hbm, out_hbm, epk0, epk1, rows0, rows1,
                   acc, gsem0, gsem1, ssem0, ssem1, rsem0, rsem1):
        c = lax.axis_index("c")
        s = lax.axis_index("s")
        wid = c * NS + s

        def fz(i, carry):
            rows0[i // dl, pl.ds((i % dl) * 16, 16)] = jnp.zeros((16,), jnp.float32)
            return carry

        lax.fori_loop(0, zrows * dl, fz, 0)
        r0 = s * rpt
        for b in range(zfull):
            pltpu.sync_copy(rows0.at[pl.ds(0, zrows), :],
                            acc.at[pl.ds(r0 + b * zrows, zrows), :])
        if zrem:
            pltpu.sync_copy(rows0.at[pl.ds(0, zrem), :],
                            acc.at[pl.ds(r0 + zfull * zrows, zrem), :])
        plsc.subcore_barrier()

        rows = (rows0, rows1)
        gsem = (gsem0, gsem1)
        ssem = (ssem0, ssem1)
        epk = (epk0, epk1)
        rsem = (rsem0, rsem1)

        pltpu.sync_copy(epk_hbm.at[wid, pl.ds(0, RF)], epk0)
        pltpu.async_copy(epk_hbm.at[wid, pl.ds(RF, RF)], epk1, rsem1)
        pltpu.async_copy(h_hbm.at[epk0.at[0, 0]], rows0, gsem0)

        @pl.loop(0, chunks, step=2 * RF)
        def _blk(j0):
            for blk in range(2):
                cur = epk[blk]
                nxt = epk[1 - blk]
                base = j0 + blk * RF
                for t in range(RF):
                    b = t % 2
                    pltpu.make_async_copy(
                        h_hbm.at[cur.at[t, 0]], rows[b], gsem[b]).wait()
                    sd = pltpu.async_copy(
                        rows[b], acc.at[cur.at[t, 1]], ssem[b], add=True)
                    if t + 1 < RF:
                        pltpu.async_copy(
                            h_hbm.at[cur.at[t + 1, 0]], rows[1 - b],
                            gsem[1 - b])
                    else:
                        @pl.when(base + RF < chunks)
                        def _():
                            pltpu.make_async_copy(
                                epk_hbm.at[wid, pl.ds(base + RF, RF)],
                                nxt, rsem[1 - blk]).wait()
                            pltpu.async_copy(
                                h_hbm.at[nxt.at[0, 0]], rows[1 - b],
                                gsem[1 - b])
                    sd.wait()

                @pl.when(base + 2 * RF < chunks)
                def _():
                    pltpu.async_copy(
                        epk_hbm.at[wid, pl.ds(base + 2 * RF, RF)],
                        cur, rsem[blk])

        plsc.subcore_barrier()
        for b in range(nfull):
            pltpu.sync_copy(acc.at[pl.ds(r0 + b * 128, 128), :],
                            out_hbm.at[c, pl.ds(r0 + b * 128, 128), :])
        if rem:
            pltpu.sync_copy(acc.at[pl.ds(r0 + nfull * 128, rem), :],
                            out_hbm.at[c, pl.ds(r0 + nfull * 128, rem), :])

    def _mm0_body(x_ref, w_ref, b_ref, o_ref):
        h = jnp.dot(x_ref[...], w_ref[...], preferred_element_type=jnp.float32)
        o_ref[...] = h + b_ref[...][None, :]

    _mm0 = pl.pallas_call(
        _mm0_body,
        grid=(grid,),
        in_specs=[
            pl.BlockSpec((rb, d), lambda i: (i, 0)),
            pl.BlockSpec((d, d), lambda i: (0, 0)),
            pl.BlockSpec((d,), lambda i: (0,)),
        ],
        out_specs=pl.BlockSpec((rb, d), lambda i: (i, 0)),
        out_shape=jax.ShapeDtypeStruct((npad, d), jnp.float32),
    )

    def _scale0_body(xw_ref, degs_ref, h_ref, dis_ref):
        deg2 = degs_ref[0, :, :1] + degs_ref[1, :, :1]
        dis2 = deg2 ** -0.5
        h_ref[...] = xw_ref[...] * dis2
        dis_ref[...] = dis2[:, 0]

    def _fc0_body(x_ref, degs_ref, w_ref, b_ref, h_ref, dis_ref):
        deg2 = degs_ref[0, :, :1] + degs_ref[1, :, :1]
        dis2 = deg2 ** -0.5
        h = jnp.dot(x_ref[...], w_ref[...], preferred_element_type=jnp.float32)
        h_ref[...] = (h + b_ref[...][None, :]) * dis2
        dis_ref[...] = dis2[:, 0]

    _fc0 = pl.pallas_call(
        _fc0_body,
        grid=(grid,),
        in_specs=[
            pl.BlockSpec((rb, d), lambda i: (i, 0)),
            pl.BlockSpec((NC, rb, DEGW), lambda i: (0, i, 0)),
            pl.BlockSpec((d, d), lambda i: (0, 0)),
            pl.BlockSpec((d,), lambda i: (0,)),
        ],
        out_specs=[
            pl.BlockSpec((rb, d), lambda i: (i, 0)),
            pl.BlockSpec((rb,), lambda i: (i,)),
        ],
        out_shape=[
            jax.ShapeDtypeStruct((npad, d), jnp.float32),
            jax.ShapeDtypeStruct((npad,), jnp.float32),
        ],
    )

    _scale0 = pl.pallas_call(
        _scale0_body,
        grid=(grid,),
        in_specs=[
            pl.BlockSpec((rb, d), lambda i: (i, 0)),
            pl.BlockSpec((NC, rb, DEGW), lambda i: (0, i, 0)),
        ],
        out_specs=[
            pl.BlockSpec((rb, d), lambda i: (i, 0)),
            pl.BlockSpec((rb,), lambda i: (i,)),
        ],
        out_shape=[
            jax.ShapeDtypeStruct((npad, d), jnp.float32),
            jax.ShapeDtypeStruct((npad,), jnp.float32),
        ],
    )

    def _fc1_body(p_ref, dis_ref, w_ref, b_ref, h_ref):
        dis2 = dis_ref[...][:, None]
        h1 = jnp.maximum(2.0 * dis2 * (p_ref[0] + p_ref[1]), 0.0)
        h = jnp.dot(h1, w_ref[...], preferred_element_type=jnp.float32)
        h_ref[...] = (h + b_ref[...][None, :]) * dis2

    _fc1 = pl.pallas_call(
        _fc1_body,
        grid=(grid,),
        in_specs=[
            pl.BlockSpec((NC, rb, d), lambda i: (0, i, 0)),
            pl.BlockSpec((rb,), lambda i: (i,)),
            pl.BlockSpec((d, d), lambda i: (0, 0)),
            pl.BlockSpec((d,), lambda i: (0,)),
        ],
        out_specs=pl.BlockSpec((rb, d), lambda i: (i, 0)),
        out_shape=jax.ShapeDtypeStruct((npad, d), jnp.float32),
    )

    def _final_body(p_ref, dis_ref, o_ref):
        o_ref[...] = (p_ref[0] + p_ref[1]) * dis_ref[...][:, None]

    _final = pl.pallas_call(
        _final_body,
        grid=(grid,),
        in_specs=[
            pl.BlockSpec((NC, rb, d), lambda i: (0, i, 0)),
            pl.BlockSpec((rb,), lambda i: (i,)),
        ],
        out_specs=pl.BlockSpec((rb, d), lambda i: (i, 0)),
        out_shape=jax.ShapeDtypeStruct((npad, d), jnp.float32),
    )

    def run(xpad, epk, w0t, b0, w1t, b1):
        degs = _degree(epk)
        h0, dis = _fc0(xpad, degs, w0t, b0)
        p0 = _propagate(h0, epk)
        h1 = _fc1(p0, dis, w1t, b1)
        p1 = _propagate(h1, epk)
        return _final(p1, dis)

    return run, npad, epad, chunks


def kernel(inputs, edge_index, W0, b0, W1, b1):
    n, d = inputs.shape
    e = edge_index.shape[1]
    run, npad, epad, chunks = _build(n, e, d)
    pad_idx = n + (jnp.arange(epad - e, dtype=jnp.int32) % (npad - n))
    src = jnp.concatenate([edge_index[0].astype(jnp.int32), pad_idx])
    dst = jnp.concatenate([edge_index[1].astype(jnp.int32), pad_idx])
    epk = jnp.stack([src.reshape(NW, chunks, K), dst.reshape(NW, chunks, K)],
                    axis=2)
    xpad = jnp.pad(inputs.astype(jnp.float32), ((0, npad - n), (0, 0)))
    out = run(xpad, epk, W0.T, b0, W1.T, b1)
    return out[:n]


# speedup vs baseline: 2.9537x; 1.0257x over previous
import functools

import jax
import jax.numpy as jnp
from jax import lax
from jax.experimental import pallas as pl
from jax.experimental.pallas import tpu as pltpu
from jax.experimental.pallas import tpu_sc as plsc

NC = 2
NS = 16
NW = NC * NS
K = 128
DEGW = 16


def _ceil_to(x, m):
    return -(-x // m) * m


@functools.lru_cache(maxsize=None)
def _build(n, e, d):
    npad = _ceil_to(n + 1, 256)
    rpt = npad // NW
    nfull, rem = divmod(rpt, 128)
    RF = 20
    epad = _ceil_to(e, NW * K * 2 * RF)
    chunks = epad // (NW * K)
    rb = max(b for b in (1024, 512, 256) if npad % b == 0)
    grid = npad // rb
    dl = d // 16

    mesh = plsc.VectorSubcoreMesh(core_axis_name="c", subcore_axis_name="s")

    @functools.partial(
        pl.kernel,
        out_type=jax.ShapeDtypeStruct((NC, npad, DEGW), jnp.float32),
        mesh=mesh,
        scratch_types=[
            pltpu.VMEM((chunks, K), jnp.int32),
            pltpu.VMEM((K, DEGW), jnp.float32),
            pltpu.VMEM((rpt, DEGW), jnp.float32),
            pltpu.VMEM_SHARED((npad, DEGW), jnp.float32),
            pltpu.SemaphoreType.DMA,
            pltpu.SemaphoreType.DMA,
            pltpu.SemaphoreType.DMA,
            pltpu.SemaphoreType.DMA,
        ],
    )
    def _degree(spk_hbm, out_hbm, spk_v, ones_v, zero_v, acc,
                sem0, sem1, sem2, sem3):
        c = lax.axis_index("c")
        s = lax.axis_index("s")
        wid = c * NS + s
        pltpu.sync_copy(spk_hbm.at[wid], spk_v)

        def fill_ones(i, carry):
            ones_v[i, :] = jnp.ones((DEGW,), jnp.float32)
            return carry

        lax.fori_loop(0, K, fill_ones, 0)

        def fill_zero(i, carry):
            zero_v[i, :] = jnp.zeros((DEGW,), jnp.float32)
            return carry

        lax.fori_loop(0, rpt, fill_zero, 0)
        r0 = s * rpt
        pltpu.sync_copy(zero_v, acc.at[pl.ds(r0, rpt), :])
        plsc.subcore_barrier()

        sems = (sem0, sem1, sem2, sem3)

        @pl.loop(0, chunks, step=4)
        def _eloop(j):
            ds_ = [pltpu.async_copy(ones_v, acc.at[spk_v.at[j + t]],
                                    sems[t], add=True) for t in range(4)]
            for dd in ds_:
                dd.wait()

        plsc.subcore_barrier()
        pltpu.sync_copy(acc.at[pl.ds(r0, rpt), :], out_hbm.at[c, pl.ds(r0, rpt), :])

    zrows = 64
    zfull, zrem = divmod(rpt, zrows)

    @functools.partial(
        pl.kernel,
        out_type=jax.ShapeDtypeStruct((NC, npad, d), jnp.float32),
        mesh=mesh,
        scratch_types=[
            pltpu.VMEM((RF, 2, K), jnp.int32),
            pltpu.VMEM((RF, 2, K), jnp.int32),
            pltpu.VMEM((K, d), jnp.float32),
            pltpu.VMEM((K, d), jnp.float32),
            pltpu.VMEM_SHARED((npad, d), jnp.float32),
            pltpu.SemaphoreType.DMA,
            pltpu.SemaphoreType.DMA,
            pltpu.SemaphoreType.DMA,
            pltpu.SemaphoreType.DMA,
            pltpu.SemaphoreType.DMA,
            pltpu.SemaphoreType.DMA,
        ],
    )
    def _propagate(h_hbm, epk_hbm, out_hbm, epk0, epk1, rows0, rows1,
                   acc, gsem0, gsem1, ssem0, ssem1, rsem0, rsem1):
        c = lax.axis_index("c")
        s = lax.axis_index("s")
        wid = c * NS + s

        def fz(i, carry):
            rows0[i // dl, pl.ds((i % dl) * 16, 16)] = jnp.zeros((16,), jnp.float32)
            return carry

        lax.fori_loop(0, zrows * dl, fz, 0)
        r0 = s * rpt
        for b in range(zfull):
            pltpu.sync_copy(rows0.at[pl.ds(0, zrows), :],
                            acc.at[pl.ds(r0 + b * zrows, zrows), :])
        if zrem:
            pltpu.sync_copy(rows0.at[pl.ds(0, zrem), :],
                            acc.at[pl.ds(r0 + zfull * zrows, zrem), :])
        plsc.subcore_barrier()

        rows = (rows0, rows1)
        gsem = (gsem0, gsem1)
        ssem = (ssem0, ssem1)
        epk = (epk0, epk1)
        rsem = (rsem0, rsem1)

        pltpu.sync_copy(epk_hbm.at[wid, pl.ds(0, RF)], epk0)
        pltpu.async_copy(epk_hbm.at[wid, pl.ds(RF, RF)], epk1, rsem1)
        pltpu.async_copy(h_hbm.at[epk0.at[0, 0]], rows0, gsem0)

        @pl.loop(0, chunks, step=2 * RF)
        def _blk(j0):
            for blk in range(2):
                cur = epk[blk]
                nxt = epk[1 - blk]
                base = j0 + blk * RF
                for t in range(RF):
                    b = t % 2
                    pltpu.make_async_copy(
                        h_hbm.at[cur.at[t, 0]], rows[b], gsem[b]).wait()
                    sd = pltpu.async_copy(
                        rows[b], acc.at[cur.at[t, 1]], ssem[b], add=True)
                    if t + 1 < RF:
                        pltpu.async_copy(
                            h_hbm.at[cur.at[t + 1, 0]], rows[1 - b],
                            gsem[1 - b])
                    else:
                        @pl.when(base + RF < chunks)
                        def _():
                            pltpu.make_async_copy(
                                epk_hbm.at[wid, pl.ds(base + RF, RF)],
                                nxt, rsem[1 - blk]).wait()
                            pltpu.async_copy(
                                h_hbm.at[nxt.at[0, 0]], rows[1 - b],
                                gsem[1 - b])
                    sd.wait()

                @pl.when(base + 2 * RF < chunks)
                def _():
                    pltpu.async_copy(
                        epk_hbm.at[wid, pl.ds(base + 2 * RF, RF)],
                        cur, rsem[blk])

        plsc.subcore_barrier()
        for b in range(nfull):
            pltpu.sync_copy(acc.at[pl.ds(r0 + b * 128, 128), :],
                            out_hbm.at[c, pl.ds(r0 + b * 128, 128), :])
        if rem:
            pltpu.sync_copy(acc.at[pl.ds(r0 + nfull * 128, rem), :],
                            out_hbm.at[c, pl.ds(r0 + nfull * 128, rem), :])

    def _mm0_body(x_ref, w_ref, b_ref, o_ref):
        h = jnp.dot(x_ref[...], w_ref[...], preferred_element_type=jnp.float32)
        o_ref[...] = h + b_ref[...][None, :]

    _mm0 = pl.pallas_call(
        _mm0_body,
        grid=(grid,),
        in_specs=[
            pl.BlockSpec((rb, d), lambda i: (i, 0)),
            pl.BlockSpec((d, d), lambda i: (0, 0)),
            pl.BlockSpec((d,), lambda i: (0,)),
        ],
        out_specs=pl.BlockSpec((rb, d), lambda i: (i, 0)),
        out_shape=jax.ShapeDtypeStruct((npad, d), jnp.float32),
    )

    def _scale0_body(xw_ref, degs_ref, h_ref, dis_ref):
        deg2 = degs_ref[0, :, :1] + degs_ref[1, :, :1]
        dis2 = deg2 ** -0.5
        h_ref[...] = xw_ref[...] * dis2
        dis_ref[...] = dis2[:, 0]

    def _fc0_body(x_ref, degs_ref, w_ref, b_ref, h_ref, dis_ref):
        deg2 = degs_ref[0, :, :1] + degs_ref[1, :, :1]
        dis2 = deg2 ** -0.5
        h = jnp.dot(x_ref[...], w_ref[...], preferred_element_type=jnp.float32)
        h_ref[...] = (h + b_ref[...][None, :]) * dis2
        dis_ref[...] = dis2[:, 0]

    _fc0 = pl.pallas_call(
        _fc0_body,
        grid=(grid,),
        in_specs=[
            pl.BlockSpec((rb, d), lambda i: (i, 0)),
            pl.BlockSpec((NC, rb, DEGW), lambda i: (0, i, 0)),
            pl.BlockSpec((d, d), lambda i: (0, 0)),
            pl.BlockSpec((d,), lambda i: (0,)),
        ],
        out_specs=[
            pl.BlockSpec((rb, d), lambda i: (i, 0)),
            pl.BlockSpec((rb,), lambda i: (i,)),
        ],
        out_shape=[
            jax.ShapeDtypeStruct((npad, d), jnp.float32),
            jax.ShapeDtypeStruct((npad,), jnp.float32),
        ],
    )

    _scale0 = pl.pallas_call(
        _scale0_body,
        grid=(grid,),
        in_specs=[
            pl.BlockSpec((rb, d), lambda i: (i, 0)),
            pl.BlockSpec((NC, rb, DEGW), lambda i: (0, i, 0)),
        ],
        out_specs=[
            pl.BlockSpec((rb, d), lambda i: (i, 0)),
            pl.BlockSpec((rb,), lambda i: (i,)),
        ],
        out_shape=[
            jax.ShapeDtypeStruct((npad, d), jnp.float32),
            jax.ShapeDtypeStruct((npad,), jnp.float32),
        ],
    )

    def _fc1_body(p_ref, dis_ref, w_ref, b_ref, h_ref):
        dis2 = dis_ref[...][:, None]
        h1 = jnp.maximum(2.0 * dis2 * (p_ref[0] + p_ref[1]), 0.0)
        h = jnp.dot(h1, w_ref[...], preferred_element_type=jnp.float32)
        h_ref[...] = (h + b_ref[...][None, :]) * dis2

    _fc1 = pl.pallas_call(
        _fc1_body,
        grid=(grid,),
        in_specs=[
            pl.BlockSpec((NC, rb, d), lambda i: (0, i, 0)),
            pl.BlockSpec((rb,), lambda i: (i,)),
            pl.BlockSpec((d, d), lambda i: (0, 0)),
            pl.BlockSpec((d,), lambda i: (0,)),
        ],
        out_specs=pl.BlockSpec((rb, d), lambda i: (i, 0)),
        out_shape=jax.ShapeDtypeStruct((npad, d), jnp.float32),
    )

    def _final_body(p_ref, dis_ref, o_ref):
        o_ref[...] = (p_ref[0] + p_ref[1]) * dis_ref[...]

    rbf = n // grid if n % grid == 0 and (n // grid) % 8 == 0 else None
    nf = n if rbf else npad
    _final = pl.pallas_call(
        _final_body,
        grid=(grid,),
        in_specs=[
            pl.BlockSpec((NC, rbf or rb, d), lambda i: (0, i, 0)),
            pl.BlockSpec((rbf or rb, 1), lambda i: (i, 0)),
        ],
        out_specs=pl.BlockSpec((rbf or rb, d), lambda i: (i, 0)),
        out_shape=jax.ShapeDtypeStruct((nf, d), jnp.float32),
    )

    def run(xpad, spk, epk, w0t, b0, w1t, b1):
        degs = _degree(spk)
        h0, dis = _fc0(xpad, degs, w0t, b0)
        p0 = _propagate(h0, epk)
        h1 = _fc1(p0, dis, w1t, b1)
        p1 = _propagate(h1, epk)
        out = _final(p1, dis[:, None])
        return out if rbf else out[:n]

    return run, npad, epad, chunks


def kernel(inputs, edge_index, W0, b0, W1, b1):
    n, d = inputs.shape
    e = edge_index.shape[1]
    run, npad, epad, chunks = _build(n, e, d)
    pad_idx = n + (jnp.arange(epad - e, dtype=jnp.int32) % (npad - n))
    src = jnp.concatenate([edge_index[0].astype(jnp.int32), pad_idx])
    dst = jnp.concatenate([edge_index[1].astype(jnp.int32), pad_idx])
    spk = src.reshape(NW, chunks, K)
    epk = jnp.stack([spk, dst.reshape(NW, chunks, K)], axis=2)
    xpad = jnp.pad(inputs.astype(jnp.float32), ((0, npad - n), (0, 0)))
    return run(xpad, spk, epk, W0.T, b0, W1.T, b1)
